# Initial kernel scaffold; baseline (speedup 1.0000x reference)
import jax
import jax.numpy as jnp
from jax.experimental import pallas as pl


def kernel(query_embed, operators_embedding, params):
    raise NotImplementedError("write your pallas kernel here")


# speedup vs baseline: 1.8842x; 1.8842x over previous
import functools

import jax
import jax.numpy as jnp
from jax.experimental import pallas as pl
from jax.experimental.pallas import tpu as pltpu

D = 2048
H = 256
L = 4
N = 4096
ROWS = 256
NTILES = N // ROWS
CHUNK = 512
EPS = 1e-12


def _matmul_kernel(ops_ref, w_ref, query_ref, qw_ref, qb_ref, e_ref, qn_ref):
    j = pl.program_id(0)

    @pl.when(j == 0)
    def _():
        qr = jnp.dot(query_ref[:], qw_ref[:],
                     preferred_element_type=jnp.float32) + qb_ref[:]
        for l in range(L):
            sl = qr[0:1, l * H:(l + 1) * H]
            nn = jnp.sum(sl * sl)
            denom = jnp.maximum(jnp.sqrt(nn), EPS)
            qn_ref[l] = sl / denom

    ops_tile = ops_ref[:]
    for l in range(L):
        e_ref[l] = jnp.dot(ops_tile, w_ref[l],
                           preferred_element_type=jnp.float32)


def _routing_kernel(e_ref, b_ref, qn_ref, ob_ref, ops_ref, out_ref,
                    scores_ref, row_ref, idx_ref, sem):
    i = pl.program_id(0)

    @pl.when(i == 0)
    def _():
        out_ref[:] = jnp.zeros_like(out_ref)

    @pl.when(i > 0)
    def _():
        prev = idx_ref[0]
        pltpu.make_async_copy(
            ops_ref.at[pl.ds(prev, 1)], row_ref, sem).start()
        pltpu.make_async_copy(
            ops_ref.at[pl.ds(prev, 1)], row_ref, sem).wait()

    qn = qn_ref[0]
    ob = ob_ref[0]
    proj = jnp.dot(row_ref[:], b_ref[0],
                   preferred_element_type=jnp.float32) + ob
    c = jnp.where(i == 0, ob, proj)

    qc = jnp.sum(qn * c)
    cc = jnp.sum(c * c)

    for k in range(N // CHUNK):
        e = e_ref[0, pl.ds(k * CHUNK, CHUNK), :]
        s = jnp.sum(e * qn, axis=1)
        nn = jnp.sum(e * e, axis=1)
        dd = jnp.sum(e * c, axis=1)
        denom = jnp.maximum(
            jnp.sqrt(jnp.maximum(nn + 2.0 * dd + cc, 0.0)), EPS)
        sc = (s + qc) / denom
        scores_ref[pl.ds(k * (CHUNK // 128), CHUNK // 128), :] = (
            sc.reshape(CHUNK // 128, 128))

    scores = scores_ref[:]
    mx = jnp.max(scores)
    se = jnp.sum(jnp.exp(scores - mx))
    s00 = scores_ref[0, 0]
    out_val = jnp.where(i == 0, s00 - mx, 0.0) - jnp.log(se)

    r_iota = jax.lax.broadcasted_iota(jnp.int32, (N // 128, 128), 0)
    l_iota = jax.lax.broadcasted_iota(jnp.int32, (N // 128, 128), 1)
    gidx = r_iota * 128 + l_iota
    amax = jnp.min(jnp.where(scores == mx, gidx, N))
    idx_ref[0] = jnp.where(i == 0, 0, amax)

    o_iota = jax.lax.broadcasted_iota(jnp.int32, (8, 128), 0)
    z_iota = jax.lax.broadcasted_iota(jnp.int32, (8, 128), 1)
    mask = (o_iota == i) & (z_iota == 0)
    out_ref[:] = jnp.where(mask, out_val, out_ref[:])


@jax.jit
def kernel(query_embed, operators_embedding, params):
    ops = operators_embedding
    w = jnp.stack([params['oW0']] +
                  [params['oW%d' % i][:D] for i in range(1, L)])
    b = jnp.stack([params['oW%d' % i][D:] for i in range(1, L)])
    qw = jnp.concatenate([params['qW%d' % i] for i in range(L)], axis=1)
    qb = jnp.concatenate([params['qb%d' % i] for i in range(L)])[None, :]
    ob = jnp.stack([params['ob%d' % i] for i in range(L)])[:, None, :]

    e, qn = pl.pallas_call(
        _matmul_kernel,
        grid=(NTILES,),
        in_specs=[
            pl.BlockSpec((ROWS, D), lambda j: (j, 0)),
            pl.BlockSpec((L, D, H), lambda j: (0, 0, 0)),
            pl.BlockSpec((1, D), lambda j: (0, 0)),
            pl.BlockSpec((D, L * H), lambda j: (0, 0)),
            pl.BlockSpec((1, L * H), lambda j: (0, 0)),
        ],
        out_specs=[
            pl.BlockSpec((L, ROWS, H), lambda j: (0, j, 0)),
            pl.BlockSpec((L, 1, H), lambda j: (0, 0, 0)),
        ],
        out_shape=[
            jax.ShapeDtypeStruct((L, N, H), jnp.float32),
            jax.ShapeDtypeStruct((L, 1, H), jnp.float32),
        ],
    )(ops, w, query_embed, qw, qb)

    out_pad = pl.pallas_call(
        _routing_kernel,
        grid=(L,),
        in_specs=[
            pl.BlockSpec((1, N, H), lambda i: (i, 0, 0)),
            pl.BlockSpec((1, D, H), lambda i: (jnp.maximum(i - 1, 0), 0, 0)),
            pl.BlockSpec((1, 1, H), lambda i: (i, 0, 0)),
            pl.BlockSpec((1, 1, H), lambda i: (i, 0, 0)),
            pl.BlockSpec(memory_space=pl.ANY),
        ],
        out_specs=pl.BlockSpec((8, 128), lambda i: (0, 0)),
        out_shape=jax.ShapeDtypeStruct((8, 128), jnp.float32),
        scratch_shapes=[
            pltpu.VMEM((N // 128, 128), jnp.float32),
            pltpu.VMEM((1, D), jnp.float32),
            pltpu.SMEM((1,), jnp.int32),
            pltpu.SemaphoreType.DMA,
        ],
    )(e, b, qn, ob, ops)

    return out_pad[:L, 0]


# speedup vs baseline: 2.2153x; 1.1757x over previous
import jax
import jax.numpy as jnp
from jax.experimental import pallas as pl
from jax.experimental.pallas import tpu as pltpu

D = 2048
H = 256
L = 4
N = 4096
ROWS = 256
NTILES = N // ROWS
CHUNK = 512
EPS = 1e-12


def _fused_kernel(ops_ref, w_ref, query_ref, qw_ref, qb_ref, b_ref, ob_ref,
                  ops_any, out_ref,
                  e_s, qn_s, scores_s, row_s, idx_s, sem):
    j = pl.program_id(0)

    @pl.when(j == 0)
    def _():
        qr = jnp.dot(query_ref[:], qw_ref[:],
                     preferred_element_type=jnp.float32) + qb_ref[:]
        for l in range(L):
            sl = qr[0:1, l * H:(l + 1) * H]
            nn = jnp.sum(sl * sl)
            denom = jnp.maximum(jnp.sqrt(nn), EPS)
            qn_s[l] = sl / denom

    @pl.when(j < NTILES)
    def _():
        ops_tile = ops_ref[:]
        for l in range(L):
            e_s[l, pl.ds(j * ROWS, ROWS), :] = jnp.dot(
                ops_tile, w_ref[l], preferred_element_type=jnp.float32)

    @pl.when(j >= NTILES)
    def _():
        i = j - NTILES

        @pl.when(i == 0)
        def _():
            out_ref[:] = jnp.zeros_like(out_ref)

        @pl.when(i > 0)
        def _():
            prev = idx_s[0]
            cp = pltpu.make_async_copy(
                ops_any.at[pl.ds(prev, 1)], row_s, sem)
            cp.start()
            cp.wait()

        qn = qn_s[pl.ds(jnp.where(i == 0, 0, i), 1)][0]
        ob = ob_ref[pl.ds(i, 1)][0]
        bw = b_ref[pl.ds(jnp.maximum(i - 1, 0), 1)][0]
        proj = jnp.dot(row_s[:], bw,
                       preferred_element_type=jnp.float32) + ob
        c = jnp.where(i == 0, ob, proj)

        qc = jnp.sum(qn * c)
        cc = jnp.sum(c * c)

        for k in range(N // CHUNK):
            e = e_s[pl.ds(i, 1), pl.ds(k * CHUNK, CHUNK), :][0]
            s = jnp.sum(e * qn, axis=1)
            nn = jnp.sum(e * e, axis=1)
            dd = jnp.sum(e * c, axis=1)
            denom = jnp.maximum(
                jnp.sqrt(jnp.maximum(nn + 2.0 * dd + cc, 0.0)), EPS)
            sc = (s + qc) / denom
            scores_s[pl.ds(k * (CHUNK // 128), CHUNK // 128), :] = (
                sc.reshape(CHUNK // 128, 128))

        scores = scores_s[:]
        mx = jnp.max(scores)
        se = jnp.sum(jnp.exp(scores - mx))
        s00 = scores_s[0, 0]
        out_val = jnp.where(i == 0, s00 - mx, 0.0) - jnp.log(se)

        r_iota = jax.lax.broadcasted_iota(jnp.int32, (N // 128, 128), 0)
        l_iota = jax.lax.broadcasted_iota(jnp.int32, (N // 128, 128), 1)
        gidx = r_iota * 128 + l_iota
        amax = jnp.min(jnp.where(scores == mx, gidx, N))
        idx_s[0] = jnp.where(i == 0, 0, amax)

        o_iota = jax.lax.broadcasted_iota(jnp.int32, (8, 128), 0)
        z_iota = jax.lax.broadcasted_iota(jnp.int32, (8, 128), 1)
        mask = (o_iota == i) & (z_iota == 0)
        out_ref[:] = jnp.where(mask, out_val, out_ref[:])


@jax.jit
def kernel(query_embed, operators_embedding, params):
    ops = operators_embedding
    w = jnp.stack([params['oW0']] +
                  [params['oW%d' % i][:D] for i in range(1, L)])
    b = jnp.stack([params['oW%d' % i][D:] for i in range(1, L)])
    qw = jnp.concatenate([params['qW%d' % i] for i in range(L)], axis=1)
    qb = jnp.concatenate([params['qb%d' % i] for i in range(L)])[None, :]
    ob = jnp.stack([params['ob%d' % i] for i in range(L)])[:, None, :]

    out_pad = pl.pallas_call(
        _fused_kernel,
        grid=(NTILES + L,),
        in_specs=[
            pl.BlockSpec((ROWS, D), lambda j: (jnp.minimum(j, NTILES - 1), 0)),
            pl.BlockSpec((L, D, H), lambda j: (0, 0, 0)),
            pl.BlockSpec((1, D), lambda j: (0, 0)),
            pl.BlockSpec((D, L * H), lambda j: (0, 0)),
            pl.BlockSpec((1, L * H), lambda j: (0, 0)),
            pl.BlockSpec((L - 1, D, H), lambda j: (0, 0, 0)),
            pl.BlockSpec((L, 1, H), lambda j: (0, 0, 0)),
            pl.BlockSpec(memory_space=pl.ANY),
        ],
        out_specs=pl.BlockSpec((8, 128), lambda j: (0, 0)),
        out_shape=jax.ShapeDtypeStruct((8, 128), jnp.float32),
        scratch_shapes=[
            pltpu.VMEM((L, N, H), jnp.float32),
            pltpu.VMEM((L, 1, H), jnp.float32),
            pltpu.VMEM((N // 128, 128), jnp.float32),
            pltpu.VMEM((1, D), jnp.float32),
            pltpu.SMEM((1,), jnp.int32),
            pltpu.SemaphoreType.DMA,
        ],
    )(ops, w, query_embed, qw, qb, b, ob, ops)

    return out_pad[:L, 0]
